# baseline (speedup 1.0000x reference)
import functools

import jax
import jax.numpy as jnp
from jax import lax
from jax.experimental import pallas as pl
from jax.experimental.pallas import tpu as pltpu
from jax.experimental.pallas import tpu_sc as plsc

NC = 2
NS = 16
NW = NC * NS
LANES = 16


@functools.partial(jax.jit, static_argnames=())
def kernel(tokens, embedding):
    B, L = tokens.shape
    V, D = embedding.shape
    assert B % NW == 0 and D % LANES == 0
    BPW = B // NW
    RND = 128
    assert (BPW * L) % RND == 0 and RND % L == 0
    ROWS_PER_RND = RND // L
    NROUNDS = (BPW * L) // RND
    NVREG = D // LANES

    tok = tokens.astype(jnp.int32).reshape(NW, NROUNDS, RND)

    mesh = plsc.VectorSubcoreMesh(core_axis_name="c", subcore_axis_name="s")

    @functools.partial(
        pl.kernel,
        out_type=jax.ShapeDtypeStruct((B, D), jnp.float32),
        mesh=mesh,
        compiler_params=pltpu.CompilerParams(use_tc_tiling_on_sc=False),
        scratch_types=[
            pltpu.VMEM((NROUNDS, RND), jnp.int32),
            pltpu.VMEM((RND, D), jnp.float32),
            pltpu.VMEM((BPW, D), jnp.float32),
            pltpu.SemaphoreType.DMA,
        ],
    )
    def run(tok_hbm, table_hbm, out_hbm, idx_v, rowbuf, out_v, sem):
        wid = lax.axis_index("s") * NC + lax.axis_index("c")
        pltpu.sync_copy(tok_hbm.at[wid], idx_v)

        inv_l = jnp.float32(1.0 / L)

        def round_body(j, carry):
            pltpu.async_copy(table_hbm.at[idx_v.at[j]], rowbuf, sem).wait()
            for r in range(ROWS_PER_RND):
                for c in range(NVREG):
                    acc = rowbuf[r * L, pl.ds(c * LANES, LANES)]
                    for k in range(1, L):
                        acc = acc + rowbuf[r * L + k, pl.ds(c * LANES, LANES)]
                    out_v[j * ROWS_PER_RND + r, pl.ds(c * LANES, LANES)] = (
                        acc * inv_l)
            return carry

        lax.fori_loop(0, NROUNDS, round_body, 0)
        pltpu.sync_copy(out_v, out_hbm.at[pl.ds(wid * BPW, BPW)])

    return run(tok, embedding)
